# baseline (speedup 1.0000x reference)
import jax
import jax.numpy as jnp
from jax.experimental import pallas as pl
from jax.experimental.pallas import tpu as pltpu

_D = 2048
_E = 16
_TOPK = 2
_ALPHA = 0.01
_BETA = 0.1
_BLK = 1024


def _router_kernel(x_ref, w_ref, b_ref, idx_ref, comb_ref, bal_ref, z_ref,
                   acc_ref):
    i = pl.program_id(0)
    n = pl.num_programs(0)

    @pl.when(i == 0)
    def _init():
        acc_ref[...] = jnp.zeros_like(acc_ref)

    logits = jnp.dot(x_ref[...], w_ref[...],
                     preferred_element_type=jnp.float32) + b_ref[...]
    m = jnp.max(logits, axis=-1, keepdims=True)
    e = jnp.exp(logits - m)
    p = e / jnp.sum(e, axis=-1, keepdims=True)

    v1 = jnp.max(p, axis=-1, keepdims=True)
    i1 = jnp.argmax(p, axis=-1)
    iota = jax.lax.broadcasted_iota(jnp.int32, p.shape, 1)
    pm = jnp.where(iota == i1[:, None], -1.0, p)
    v2 = jnp.max(pm, axis=-1, keepdims=True)
    i2 = jnp.argmax(pm, axis=-1)
    denom = v1 + v2

    idx_ref[...] = jnp.concatenate([i1[:, None], i2[:, None]], axis=1)
    comb_ref[...] = jnp.concatenate([v1 / denom, v2 / denom], axis=1)

    is_max = (p == v1).astype(jnp.float32)
    acc_ref[0:1, :] += jnp.sum(is_max, axis=0, keepdims=True)
    acc_ref[1:2, :] += jnp.sum(p, axis=0, keepdims=True)
    lse = jnp.log(jnp.sum(jnp.exp(p), axis=-1))
    acc_ref[2:3, :] += jnp.sum(jnp.reshape(lse * lse, (_BLK // _E, _E)),
                               axis=0, keepdims=True)

    @pl.when(i == n - 1)
    def _finalize():
        ntok = jnp.float32(n * _BLK)
        f = acc_ref[0:1, :] / ntok
        cap = acc_ref[1:2, :] / ntok
        bal = _ALPHA * jnp.sum(f * cap) / _E
        z = _BETA * jnp.sum(acc_ref[2:3, :]) / ntok
        bal_ref[...] = jnp.full(bal_ref.shape, bal, jnp.float32)
        z_ref[...] = jnp.full(z_ref.shape, z, jnp.float32)


def kernel(x, W, b):
    B, T, D = x.shape
    N = B * T
    xr = x.reshape(N, D)
    b2 = b.reshape(1, _E).astype(jnp.float32)
    grid = (N // _BLK,)

    idx, comb, bal, z = pl.pallas_call(
        _router_kernel,
        grid=grid,
        in_specs=[
            pl.BlockSpec((_BLK, D), lambda i: (i, 0)),
            pl.BlockSpec((D, _E), lambda i: (0, 0)),
            pl.BlockSpec((1, _E), lambda i: (0, 0)),
        ],
        out_specs=[
            pl.BlockSpec((_BLK, _TOPK), lambda i: (i, 0)),
            pl.BlockSpec((_BLK, _TOPK), lambda i: (i, 0)),
            pl.BlockSpec((1, 128), lambda i: (0, 0)),
            pl.BlockSpec((1, 128), lambda i: (0, 0)),
        ],
        out_shape=[
            jax.ShapeDtypeStruct((N, _TOPK), jnp.int32),
            jax.ShapeDtypeStruct((N, _TOPK), jnp.float32),
            jax.ShapeDtypeStruct((1, 128), jnp.float32),
            jax.ShapeDtypeStruct((1, 128), jnp.float32),
        ],
        scratch_shapes=[pltpu.VMEM((3, _E), jnp.float32)],
        compiler_params=pltpu.CompilerParams(
            dimension_semantics=("arbitrary",)),
    )(xr, W, b2)

    topk_indices = idx.reshape(B, T, _TOPK)
    combine_scores = comb.reshape(B, T, _TOPK)
    balance_loss = bal[0, 0].reshape(())
    z_routing_loss = z[0, 0].reshape(())
    return topk_indices, combine_scores, balance_loss, z_routing_loss


# speedup vs baseline: 1.0417x; 1.0417x over previous
import jax
import jax.numpy as jnp
from jax.experimental import pallas as pl
from jax.experimental.pallas import tpu as pltpu

_D = 2048
_E = 16
_TOPK = 2
_ALPHA = 0.01
_BETA = 0.1
_BLK = 2048


def _router_kernel(x_ref, w_ref, b_ref, idx_ref, comb_ref, bal_ref, z_ref,
                   acc_ref):
    i = pl.program_id(0)
    n = pl.num_programs(0)

    @pl.when(i == 0)
    def _init():
        acc_ref[...] = jnp.zeros_like(acc_ref)

    logits = jnp.dot(x_ref[...], w_ref[...],
                     preferred_element_type=jnp.float32) + b_ref[...]
    m = jnp.max(logits, axis=-1, keepdims=True)
    e = jnp.exp(logits - m)
    p = e / jnp.sum(e, axis=-1, keepdims=True)

    v1 = jnp.max(p, axis=-1, keepdims=True)
    i1 = jnp.argmax(p, axis=-1)
    iota = jax.lax.broadcasted_iota(jnp.int32, p.shape, 1)
    pm = jnp.where(iota == i1[:, None], -1.0, p)
    v2 = jnp.max(pm, axis=-1, keepdims=True)
    i2 = jnp.argmax(pm, axis=-1)
    denom = v1 + v2

    idx_ref[...] = jnp.concatenate([i1[:, None], i2[:, None]], axis=1)
    comb_ref[...] = jnp.concatenate([v1 / denom, v2 / denom], axis=1)

    is_max = (p == v1).astype(jnp.float32)
    acc_ref[0:1, :] += jnp.sum(is_max, axis=0, keepdims=True)
    acc_ref[1:2, :] += jnp.sum(p, axis=0, keepdims=True)
    lse = jnp.log(jnp.sum(jnp.exp(p), axis=-1))
    acc_ref[2:3, :] += jnp.sum(jnp.reshape(lse * lse, (_BLK // _E, _E)),
                               axis=0, keepdims=True)

    @pl.when(i == n - 1)
    def _finalize():
        ntok = jnp.float32(n * _BLK)
        f = acc_ref[0:1, :] / ntok
        cap = acc_ref[1:2, :] / ntok
        bal = _ALPHA * jnp.sum(f * cap) / _E
        z = _BETA * jnp.sum(acc_ref[2:3, :]) / ntok
        bal_ref[...] = jnp.full(bal_ref.shape, bal, jnp.float32)
        z_ref[...] = jnp.full(z_ref.shape, z, jnp.float32)


def kernel(x, W, b):
    B, T, D = x.shape
    N = B * T
    xr = x.reshape(N, D)
    b2 = b.reshape(1, _E).astype(jnp.float32)
    grid = (N // _BLK,)

    idx, comb, bal, z = pl.pallas_call(
        _router_kernel,
        grid=grid,
        in_specs=[
            pl.BlockSpec((_BLK, D), lambda i: (i, 0)),
            pl.BlockSpec((D, _E), lambda i: (0, 0)),
            pl.BlockSpec((1, _E), lambda i: (0, 0)),
        ],
        out_specs=[
            pl.BlockSpec((_BLK, _TOPK), lambda i: (i, 0)),
            pl.BlockSpec((_BLK, _TOPK), lambda i: (i, 0)),
            pl.BlockSpec((1, 128), lambda i: (0, 0)),
            pl.BlockSpec((1, 128), lambda i: (0, 0)),
        ],
        out_shape=[
            jax.ShapeDtypeStruct((N, _TOPK), jnp.int32),
            jax.ShapeDtypeStruct((N, _TOPK), jnp.float32),
            jax.ShapeDtypeStruct((1, 128), jnp.float32),
            jax.ShapeDtypeStruct((1, 128), jnp.float32),
        ],
        scratch_shapes=[pltpu.VMEM((3, _E), jnp.float32)],
        compiler_params=pltpu.CompilerParams(
            dimension_semantics=("arbitrary",)),
    )(xr, W, b2)

    topk_indices = idx.reshape(B, T, _TOPK)
    combine_scores = comb.reshape(B, T, _TOPK)
    balance_loss = bal[0, 0].reshape(())
    z_routing_loss = z[0, 0].reshape(())
    return topk_indices, combine_scores, balance_loss, z_routing_loss


# speedup vs baseline: 1.4085x; 1.3521x over previous
import jax
import jax.numpy as jnp
from jax.experimental import pallas as pl
from jax.experimental.pallas import tpu as pltpu

_D = 2048
_E = 16
_TOPK = 2
_ALPHA = 0.01
_BETA = 0.1
_BLK = 1024


def _router_kernel(x_ref, w_ref, b_ref, idx_ref, comb_ref, bal_ref, z_ref,
                   acc_ref):
    i = pl.program_id(0)
    n = pl.num_programs(0)

    @pl.when(i == 0)
    def _init():
        acc_ref[...] = jnp.zeros_like(acc_ref)

    logits = jnp.dot(x_ref[...], w_ref[...],
                     preferred_element_type=jnp.float32)
    lt = logits.T + b_ref[...]
    m = jnp.max(lt, axis=0, keepdims=True)
    e = jnp.exp(lt - m)
    p = e / jnp.sum(e, axis=0, keepdims=True)

    iota = jax.lax.broadcasted_iota(jnp.int32, p.shape, 0)
    v1 = jnp.max(p, axis=0, keepdims=True)
    i1 = jnp.min(jnp.where(p == v1, iota, _E), axis=0, keepdims=True)
    pm = jnp.where(iota == i1, -1.0, p)
    v2 = jnp.max(pm, axis=0, keepdims=True)
    i2 = jnp.min(jnp.where(pm == v2, iota, _E), axis=0, keepdims=True)
    denom = v1 + v2

    idx_ref[...] = jnp.concatenate([i1, i2], axis=0)
    comb_ref[...] = jnp.concatenate([v1 / denom, v2 / denom], axis=0)

    is_max = (p == v1).astype(jnp.float32)
    acc_ref[:, 0:1] += jnp.sum(is_max, axis=1, keepdims=True)
    acc_ref[:, 1:2] += jnp.sum(p, axis=1, keepdims=True)
    lse = jnp.log(jnp.sum(jnp.exp(p), axis=0, keepdims=True))
    acc_ref[0:1, 2:3] += jnp.sum(lse * lse, axis=1, keepdims=True)

    @pl.when(i == n - 1)
    def _finalize():
        ntok = jnp.float32(n * _BLK)
        f = acc_ref[:, 0:1] / ntok
        cap = acc_ref[:, 1:2] / ntok
        bal = _ALPHA * jnp.sum(f * cap, axis=0, keepdims=True) / _E
        z = _BETA * acc_ref[0:1, 2:3] / ntok
        bal_ref[...] = jnp.broadcast_to(bal, bal_ref.shape)
        z_ref[...] = jnp.broadcast_to(z, z_ref.shape)


def kernel(x, W, b):
    B, T, D = x.shape
    N = B * T
    xr = x.reshape(N, D)
    b2 = b.reshape(_E, 1).astype(jnp.float32)
    grid = (N // _BLK,)

    idx, comb, bal, z = pl.pallas_call(
        _router_kernel,
        grid=grid,
        in_specs=[
            pl.BlockSpec((_BLK, D), lambda i: (i, 0)),
            pl.BlockSpec((D, _E), lambda i: (0, 0)),
            pl.BlockSpec((_E, 1), lambda i: (0, 0)),
        ],
        out_specs=[
            pl.BlockSpec((_TOPK, _BLK), lambda i: (0, i)),
            pl.BlockSpec((_TOPK, _BLK), lambda i: (0, i)),
            pl.BlockSpec((1, 128), lambda i: (0, 0)),
            pl.BlockSpec((1, 128), lambda i: (0, 0)),
        ],
        out_shape=[
            jax.ShapeDtypeStruct((_TOPK, N), jnp.int32),
            jax.ShapeDtypeStruct((_TOPK, N), jnp.float32),
            jax.ShapeDtypeStruct((1, 128), jnp.float32),
            jax.ShapeDtypeStruct((1, 128), jnp.float32),
        ],
        scratch_shapes=[pltpu.VMEM((_E, 128), jnp.float32)],
        compiler_params=pltpu.CompilerParams(
            dimension_semantics=("arbitrary",)),
    )(xr, W, b2)

    topk_indices = idx.T.reshape(B, T, _TOPK)
    combine_scores = comb.T.reshape(B, T, _TOPK)
    balance_loss = bal[0, 0].reshape(())
    z_routing_loss = z[0, 0].reshape(())
    return topk_indices, combine_scores, balance_loss, z_routing_loss
